# baseline (speedup 1.0000x reference)
import functools

import jax
import jax.numpy as jnp
from jax import lax
from jax.experimental import pallas as pl
from jax.experimental.pallas import tpu as pltpu
from jax.experimental.pallas import tpu_sc as plsc

NSEG = 16
ROWS = 32768
D = 512
LANES = 16
VPR = D // LANES
NC = 2
NS = 16
NW = NC * NS

TC_ROWS = 20480
SC_ROWS = ROWS - TC_ROWS
RPW = SC_ROWS // NW
CHUNK = 64
NCHUNK = RPW // CHUNK
TBLK = 2048
NBLK = TC_ROWS // TBLK


@functools.partial(
    pl.kernel,
    out_type=(
        jax.ShapeDtypeStruct((NW, NSEG, D), jnp.float32),
        jax.ShapeDtypeStruct((NW, NSEG), jnp.float32),
    ),
    mesh=plsc.VectorSubcoreMesh(core_axis_name="c", subcore_axis_name="s"),
    compiler_params=pltpu.CompilerParams(needs_layout_passes=False),
    scratch_types=[
        pltpu.VMEM((CHUNK, D), jnp.float32),
        pltpu.VMEM((CHUNK, D), jnp.float32),
        pltpu.VMEM((RPW,), jnp.int32),
        pltpu.VMEM((NSEG, D), jnp.float32),
        pltpu.VMEM((NSEG,), jnp.float32),
        pltpu.VMEM((NSEG,), jnp.int32),
        pltpu.SemaphoreType.DMA,
        pltpu.SemaphoreType.DMA,
    ],
)
def _sc_segment_sum(x_hbm, seg_hbm,
                    sums_hbm, cnt_hbm,
                    buf0_v, buf1_v, ids_v, acc_v, cntf_v, cnti_v,
                    sem0, sem1):
    c = lax.axis_index("c")
    s = lax.axis_index("s")
    w = c * NS + s
    base = TC_ROWS + w * RPW

    pltpu.sync_copy(seg_hbm.at[pl.ds(base, RPW)], ids_v)

    zeros_i = jnp.zeros((LANES,), jnp.int32)
    ones_i = jnp.ones((LANES,), jnp.int32)
    zeros_f = jnp.zeros((LANES,), jnp.float32)
    iota = lax.iota(jnp.int32, LANES)

    for g in range(NSEG):
        for k in range(VPR):
            acc_v[g, pl.ds(k * LANES, LANES)] = zeros_f

    cnti_v[...] = zeros_i
    for i in range(RPW // LANES):
        v = ids_v[pl.ds(i * LANES, LANES)]
        plsc.addupdate_scatter(cnti_v, [v], ones_i)

    counts = cnti_v[...]
    incl = plsc.cumsum(counts)
    excl = incl - counts

    def chunk_slice(j):
        return x_hbm.at[pl.ds(base + j * CHUNK, CHUNK)]

    def process(row0, buf):
        hi_row = row0 + CHUNK
        c_end = plsc.all_reduce_population_count(incl > row0)
        c_start = plsc.all_reduce_population_count(excl < hi_row)
        g_lo = NSEG - lax.reduce_max(c_end, axes=(0,))
        g_hi = lax.reduce_max(c_start, axes=(0,))

        def seg_body(g, _):
            sel = iota == g
            start_g = lax.reduce_max(jnp.where(sel, excl, 0), axes=(0,))
            end_g = lax.reduce_max(jnp.where(sel, incl, 0), axes=(0,))
            lo = jnp.maximum(start_g - row0, 0)
            hi = jnp.minimum(end_g - row0, CHUNK)
            n = hi - lo
            half = n >> 1

            def row2(r, carry):
                r0 = lo + 2 * r
                return tuple(
                    carry[k]
                    + buf[r0, pl.ds(k * LANES, LANES)]
                    + buf[r0 + 1, pl.ds(k * LANES, LANES)]
                    for k in range(VPR))

            acc = lax.fori_loop(0, half, row2, (zeros_f,) * VPR)

            last = jnp.maximum(hi - 1, 0)
            odd = (n & 1) == 1
            for k in range(VPR):
                x_last = buf[last, pl.ds(k * LANES, LANES)]
                total = acc[k] + jnp.where(odd, x_last, 0.0)
                dst = pl.ds(k * LANES, LANES)
                acc_v[g, dst] = acc_v[g, dst] + total
            return 0

        lax.fori_loop(g_lo, g_hi, seg_body, 0)

    pltpu.async_copy(chunk_slice(0), buf0_v, sem0)

    def body2(t, _):
        j0 = 2 * t
        pltpu.async_copy(chunk_slice(j0 + 1), buf1_v, sem1)
        pltpu.make_async_copy(chunk_slice(j0), buf0_v, sem0).wait()
        process(j0 * CHUNK, buf0_v)

        @pl.when(j0 + 2 < NCHUNK)
        def _():
            pltpu.async_copy(chunk_slice(j0 + 2), buf0_v, sem0)

        pltpu.make_async_copy(chunk_slice(j0 + 1), buf1_v, sem1).wait()
        process((j0 + 1) * CHUNK, buf1_v)
        return 0

    lax.fori_loop(0, NCHUNK // 2, body2, 0)

    if NCHUNK % 2:
        pltpu.make_async_copy(chunk_slice(NCHUNK - 1), buf0_v, sem0).wait()
        process((NCHUNK - 1) * CHUNK, buf0_v)

    cntf_v[...] = counts.astype(jnp.float32)
    pltpu.sync_copy(acc_v, sums_hbm.at[w])
    pltpu.sync_copy(cntf_v, cnt_hbm.at[w])


def _tc_body(ids_ref, x_ref, sums_ref, cnt_ref):
    @pl.when(pl.program_id(0) == 0)
    def _():
        sums_ref[...] = jnp.zeros_like(sums_ref)
        cnt_ref[...] = jnp.zeros_like(cnt_ref)

    ids = ids_ref[0]
    seg_iota = lax.broadcasted_iota(jnp.int32, (NSEG, 1), 0)
    onehot = (ids == seg_iota).astype(jnp.float32)
    dn = (((1,), (0,)), ((), ()))
    sums_ref[...] += lax.dot_general(
        onehot, x_ref[...], dn,
        precision=lax.Precision.HIGHEST,
        preferred_element_type=jnp.float32)
    ones = jnp.ones((TBLK, 128), jnp.float32)
    cnt_ref[...] += lax.dot_general(
        onehot, ones, dn,
        precision=lax.Precision.HIGHEST,
        preferred_element_type=jnp.float32)


_tc_segment_sum = pl.pallas_call(
    _tc_body,
    grid=(NBLK,),
    in_specs=[
        pl.BlockSpec((1, 1, TBLK), lambda i: (i, 0, 0)),
        pl.BlockSpec((TBLK, D), lambda i: (i, 0)),
    ],
    out_specs=[
        pl.BlockSpec((NSEG, D), lambda i: (0, 0)),
        pl.BlockSpec((NSEG, 128), lambda i: (0, 0)),
    ],
    out_shape=[
        jax.ShapeDtypeStruct((NSEG, D), jnp.float32),
        jax.ShapeDtypeStruct((NSEG, 128), jnp.float32),
    ],
)


def _finish_body(sc_sums_ref, sc_cnt_ref, tc_sums_ref, tc_cnt_ref, out_ref):
    total = jnp.sum(sc_sums_ref[...], axis=0) + tc_sums_ref[...]
    cnt = jnp.sum(sc_cnt_ref[...], axis=0) + tc_cnt_ref[:, 0]
    out_ref[...] = total / jnp.maximum(cnt[:, None], 1.0)


_finish = pl.pallas_call(
    _finish_body,
    out_shape=jax.ShapeDtypeStruct((NSEG, D), jnp.float32),
)


def kernel(input_features, segment_ids):
    seg = segment_ids.astype(jnp.int32)
    sc_sums, sc_cnts = _sc_segment_sum(input_features, seg)
    tc_ids = seg[:TC_ROWS].reshape(NBLK, 1, TBLK)
    tc_sums, tc_cnt = _tc_segment_sum(tc_ids, input_features)
    return _finish(sc_sums, sc_cnts, tc_sums, tc_cnt)


# speedup vs baseline: 1.0284x; 1.0284x over previous
import functools

import jax
import jax.numpy as jnp
from jax import lax
from jax.experimental import pallas as pl
from jax.experimental.pallas import tpu as pltpu
from jax.experimental.pallas import tpu_sc as plsc

NSEG = 16
ROWS = 32768
D = 512
LANES = 16
VPR = D // LANES
NC = 2
NS = 16
NW = NC * NS

TC_ROWS = 18432
SC_ROWS = ROWS - TC_ROWS
RPW = SC_ROWS // NW
CHUNK = 64
NCHUNK = RPW // CHUNK
TBLK = 2048
NBLK = TC_ROWS // TBLK


@functools.partial(
    pl.kernel,
    out_type=(
        jax.ShapeDtypeStruct((NW, NSEG, D), jnp.float32),
        jax.ShapeDtypeStruct((NW, NSEG), jnp.float32),
    ),
    mesh=plsc.VectorSubcoreMesh(core_axis_name="c", subcore_axis_name="s"),
    compiler_params=pltpu.CompilerParams(needs_layout_passes=False),
    scratch_types=[
        pltpu.VMEM((CHUNK, D), jnp.float32),
        pltpu.VMEM((CHUNK, D), jnp.float32),
        pltpu.VMEM((RPW,), jnp.int32),
        pltpu.VMEM((NSEG, D), jnp.float32),
        pltpu.VMEM((NSEG,), jnp.float32),
        pltpu.VMEM((NSEG,), jnp.int32),
        pltpu.SemaphoreType.DMA,
        pltpu.SemaphoreType.DMA,
    ],
)
def _sc_segment_sum(x_hbm, seg_hbm,
                    sums_hbm, cnt_hbm,
                    buf0_v, buf1_v, ids_v, acc_v, cntf_v, cnti_v,
                    sem0, sem1):
    c = lax.axis_index("c")
    s = lax.axis_index("s")
    w = c * NS + s
    base = TC_ROWS + w * RPW

    def chunk_slice(j):
        return x_hbm.at[pl.ds(base + j * CHUNK, CHUNK)]

    pltpu.async_copy(chunk_slice(0), buf0_v, sem0)
    pltpu.async_copy(chunk_slice(1), buf1_v, sem1)

    pltpu.sync_copy(seg_hbm.at[pl.ds(base, RPW)], ids_v)

    zeros_i = jnp.zeros((LANES,), jnp.int32)
    ones_i = jnp.ones((LANES,), jnp.int32)
    zeros_f = jnp.zeros((LANES,), jnp.float32)
    iota = lax.iota(jnp.int32, LANES)

    for g in range(NSEG):
        for k in range(VPR):
            acc_v[g, pl.ds(k * LANES, LANES)] = zeros_f

    cnti_v[...] = zeros_i
    for i in range(RPW // LANES):
        v = ids_v[pl.ds(i * LANES, LANES)]
        plsc.addupdate_scatter(cnti_v, [v], ones_i)

    counts = cnti_v[...]
    incl = plsc.cumsum(counts)
    excl = incl - counts

    def process(row0, buf):
        hi_row = row0 + CHUNK
        c_end = plsc.all_reduce_population_count(incl > row0)
        c_start = plsc.all_reduce_population_count(excl < hi_row)
        g_lo = NSEG - lax.reduce_max(c_end, axes=(0,))
        g_hi = lax.reduce_max(c_start, axes=(0,))

        def seg_body(g, _):
            sel = iota == g
            start_g = lax.reduce_max(jnp.where(sel, excl, 0), axes=(0,))
            end_g = lax.reduce_max(jnp.where(sel, incl, 0), axes=(0,))
            lo = jnp.maximum(start_g - row0, 0)
            hi = jnp.minimum(end_g - row0, CHUNK)
            n = hi - lo
            half = n >> 1

            def row2(r, carry):
                r0 = lo + 2 * r
                return tuple(
                    carry[k]
                    + buf[r0, pl.ds(k * LANES, LANES)]
                    + buf[r0 + 1, pl.ds(k * LANES, LANES)]
                    for k in range(VPR))

            acc = lax.fori_loop(0, half, row2, (zeros_f,) * VPR)

            last = jnp.maximum(hi - 1, 0)
            odd = (n & 1) == 1
            for k in range(VPR):
                x_last = buf[last, pl.ds(k * LANES, LANES)]
                total = acc[k] + jnp.where(odd, x_last, 0.0)
                dst = pl.ds(k * LANES, LANES)
                acc_v[g, dst] = acc_v[g, dst] + total
            return 0

        lax.fori_loop(g_lo, g_hi, seg_body, 0)

    def body2(t, _):
        j0 = 2 * t
        pltpu.make_async_copy(chunk_slice(j0), buf0_v, sem0).wait()
        process(j0 * CHUNK, buf0_v)

        @pl.when(j0 + 2 < NCHUNK)
        def _():
            pltpu.async_copy(chunk_slice(j0 + 2), buf0_v, sem0)

        pltpu.make_async_copy(chunk_slice(j0 + 1), buf1_v, sem1).wait()
        process((j0 + 1) * CHUNK, buf1_v)

        @pl.when(j0 + 3 < NCHUNK)
        def _():
            pltpu.async_copy(chunk_slice(j0 + 3), buf1_v, sem1)
        return 0

    lax.fori_loop(0, NCHUNK // 2, body2, 0)

    if NCHUNK % 2:
        pltpu.make_async_copy(chunk_slice(NCHUNK - 1), buf0_v, sem0).wait()
        process((NCHUNK - 1) * CHUNK, buf0_v)

    cntf_v[...] = counts.astype(jnp.float32)
    pltpu.sync_copy(acc_v, sums_hbm.at[w])
    pltpu.sync_copy(cntf_v, cnt_hbm.at[w])


def _tc_body(ids_ref, x_ref, sums_ref, cnt_ref):
    @pl.when(pl.program_id(0) == 0)
    def _():
        sums_ref[...] = jnp.zeros_like(sums_ref)
        cnt_ref[...] = jnp.zeros_like(cnt_ref)

    ids = ids_ref[0]
    seg_iota = lax.broadcasted_iota(jnp.int32, (NSEG, 1), 0)
    onehot = (ids == seg_iota).astype(jnp.float32)
    dn = (((1,), (0,)), ((), ()))
    sums_ref[...] += lax.dot_general(
        onehot, x_ref[...], dn,
        precision=lax.Precision.HIGHEST,
        preferred_element_type=jnp.float32)
    ones = jnp.ones((TBLK, 128), jnp.float32)
    cnt_ref[...] += lax.dot_general(
        onehot, ones, dn,
        precision=lax.Precision.HIGHEST,
        preferred_element_type=jnp.float32)


_tc_segment_sum = pl.pallas_call(
    _tc_body,
    grid=(NBLK,),
    in_specs=[
        pl.BlockSpec((1, 1, TBLK), lambda i: (i, 0, 0)),
        pl.BlockSpec((TBLK, D), lambda i: (i, 0)),
    ],
    out_specs=[
        pl.BlockSpec((NSEG, D), lambda i: (0, 0)),
        pl.BlockSpec((NSEG, 128), lambda i: (0, 0)),
    ],
    out_shape=[
        jax.ShapeDtypeStruct((NSEG, D), jnp.float32),
        jax.ShapeDtypeStruct((NSEG, 128), jnp.float32),
    ],
)


def _finish_body(sc_sums_ref, sc_cnt_ref, tc_sums_ref, tc_cnt_ref, out_ref):
    total = jnp.sum(sc_sums_ref[...], axis=0) + tc_sums_ref[...]
    cnt = jnp.sum(sc_cnt_ref[...], axis=0) + tc_cnt_ref[:, 0]
    out_ref[...] = total / jnp.maximum(cnt[:, None], 1.0)


_finish = pl.pallas_call(
    _finish_body,
    out_shape=jax.ShapeDtypeStruct((NSEG, D), jnp.float32),
)


def kernel(input_features, segment_ids):
    seg = segment_ids.astype(jnp.int32)
    sc_sums, sc_cnts = _sc_segment_sum(input_features, seg)
    tc_ids = seg[:TC_ROWS].reshape(NBLK, 1, TBLK)
    tc_sums, tc_cnt = _tc_segment_sum(tc_ids, input_features)
    return _finish(sc_sums, sc_cnts, tc_sums, tc_cnt)
